# baseline (speedup 1.0000x reference)
import functools

import jax
import jax.numpy as jnp
from jax import lax
from jax.experimental import pallas as pl
from jax.experimental.pallas import tpu as pltpu
from jax.experimental.pallas import tpu_sc as plsc

_MASK_RATIO = 0.5
_Q = 4


def _sc_route_body(x_hbm, noise_hbm, out_hbm, mask_hbm, row2, mvec, pbr,
                   rkbuf, zbuf, bufa, bufb, sem_z, sem_ia, sem_oa, sem_ib,
                   sem_ob, *, n, sl, num_masked, num_cores, lanes):
    wid = lax.axis_index("s") * num_cores + lax.axis_index("c")
    base = wid * n
    one = jnp.float32(1.0)
    zero = jnp.float32(0.0)
    iota = lax.iota(jnp.int32, lanes)
    zvec = (iota * 0).astype(jnp.float32)
    for i in range(sl // lanes):
        zbuf[pl.ds(i * lanes, lanes)] = zvec
    pltpu.sync_copy(noise_hbm.at[pl.ds(base, n)], row2.at[pl.ds(0, n)])
    pltpu.sync_copy(noise_hbm.at[pl.ds(base, n)], row2.at[pl.ds(n, n)])
    for k in range(n // lanes):
        tgt = row2[pl.ds(k * lanes, lanes)]
        rank = jnp.where(row2[pl.ds(k * lanes + 1, lanes)] < tgt, one, zero)
        for s in range(2, n):
            w = row2[pl.ds(k * lanes + s, lanes)]
            rank = rank + jnp.where(w < tgt, one, zero)
        mvec[pl.ds(k * lanes, lanes)] = jnp.where(
            rank < jnp.float32(num_masked), jnp.int32(1), jnp.int32(0))
        rkbuf[pl.ds(k * lanes, lanes)] = rank
        rkbuf[pl.ds(n + k * lanes, lanes)] = rank
    pltpu.sync_copy(mvec, mask_hbm.at[pl.ds(base, n)])
    for g in range(n // lanes):
        target = (iota + jnp.int32(g * lanes)).astype(jnp.float32)
        acc = iota * 0
        for s in range(n):
            rkw = rkbuf[pl.ds(g * lanes + s, lanes)]
            jpos = (iota + jnp.int32(g * lanes + s)) & jnp.int32(n - 1)
            acc = acc + jnp.where(rkw == target, jpos, jnp.int32(0))
        pbr[pl.ds(g * lanes, lanes)] = acc

    posv = [pbr[pl.ds(g * lanes, lanes)] for g in range(n // lanes)]

    def _pos(c):
        return posv[c // lanes][c % lanes]

    for c in range(num_masked):
        off = (base + _pos(c)) * sl
        pltpu.async_copy(zbuf, out_hbm.at[pl.ds(off, sl)], sem_z)

    n_keep = n - num_masked
    quads = n_keep // _Q
    bufs = (bufa, bufb)
    sin = (sem_ia, sem_ib)
    sout = (sem_oa, sem_ob)

    def _start_in(q):
        g = q % 2
        for s in range(_Q):
            off = (base + _pos(num_masked + q * _Q + s)) * sl
            pltpu.async_copy(x_hbm.at[pl.ds(off, sl)],
                             bufs[g].at[pl.ds(s * sl, sl)], sin[g])

    def _drain(sem, count):
        for _ in range(count):
            pltpu.make_async_copy(x_hbm.at[pl.ds(0, sl)],
                                  zbuf, sem).wait()

    _start_in(0)
    for q in range(quads):
        g = q % 2
        if q + 1 < quads:
            if q >= 1:
                _drain(sout[1 - g], _Q)
            _start_in(q + 1)
        _drain(sin[g], _Q)
        for s in range(_Q):
            off = (base + _pos(num_masked + q * _Q + s)) * sl
            pltpu.async_copy(bufs[g].at[pl.ds(s * sl, sl)],
                             out_hbm.at[pl.ds(off, sl)], sout[g])
    _drain(sout[0], _Q)
    _drain(sout[1], _Q)
    _drain(sem_z, num_masked)


def _make_sc_route(b, n, sl, num_masked):
    info = plsc.get_sparse_core_info()
    num_cores, num_subcores, lanes = (
        info.num_cores, info.num_subcores, info.num_lanes)
    assert b == num_cores * num_subcores and n % lanes == 0
    mesh = plsc.VectorSubcoreMesh(core_axis_name="c", subcore_axis_name="s")
    return functools.partial(
        pl.kernel,
        out_type=[
            jax.ShapeDtypeStruct((b * n * sl,), jnp.float32),
            jax.ShapeDtypeStruct((b * n,), jnp.int32),
        ],
        mesh=mesh,
        scratch_types=[
            pltpu.VMEM((2 * n,), jnp.float32),
            pltpu.VMEM((n,), jnp.int32),
            pltpu.VMEM((n,), jnp.int32),
            pltpu.VMEM((2 * n,), jnp.float32),
            pltpu.VMEM((sl,), jnp.float32),
            pltpu.VMEM((_Q * sl,), jnp.float32),
            pltpu.VMEM((_Q * sl,), jnp.float32),
            pltpu.SemaphoreType.DMA,
            pltpu.SemaphoreType.DMA,
            pltpu.SemaphoreType.DMA,
            pltpu.SemaphoreType.DMA,
            pltpu.SemaphoreType.DMA,
        ],
    )(functools.partial(
        _sc_route_body, n=n, sl=sl, num_masked=num_masked,
        num_cores=num_cores, lanes=lanes))


def _bool_kernel(mi_ref, mb_ref):
    mb_ref[...] = mi_ref[...] != 0


def kernel(x):
    b, n, p, d = x.shape
    sl = p * d
    num_masked = int(_MASK_RATIO * n)
    noise = jax.random.uniform(jax.random.key(1), (b, n), dtype=jnp.float32)

    out_flat, mask_i32 = _make_sc_route(b, n, sl, num_masked)(
        x.reshape(b * n * sl), noise.reshape(b * n))

    mask_bool = pl.pallas_call(
        _bool_kernel,
        out_shape=jax.ShapeDtypeStruct((b, n), jnp.bool_),
    )(mask_i32.reshape(b, n))
    return out_flat.reshape(b, n, p, d), mask_bool


# speedup vs baseline: 4.1231x; 4.1231x over previous
import functools

import jax
import jax.numpy as jnp
from jax import lax
from jax.experimental import pallas as pl

_MASK_RATIO = 0.5


def _mask_kernel(noise_ref, x_ref, out_ref, mask_ref, *, n, num_masked):
    a = noise_ref[:, 0, :]
    ai = a[:, :, None]
    aj = a[:, None, :]
    bc = a.shape[0]
    ii = lax.broadcasted_iota(jnp.int32, (bc, n, n), 1)
    jj = lax.broadcasted_iota(jnp.int32, (bc, n, n), 2)
    before = (aj < ai) | ((aj == ai) & (jj < ii))
    rank = jnp.sum(before.astype(jnp.int32), axis=2)
    masked = rank < num_masked
    mask_ref[...] = masked.astype(jnp.int32)[:, None, :]
    out_ref[...] = jnp.where(masked[:, :, None, None], 0.0, x_ref[...])


def kernel(x):
    b, n, p, d = x.shape
    num_masked = int(_MASK_RATIO * n)
    bc = 4
    noise = jax.random.uniform(jax.random.key(1), (b, n), dtype=jnp.float32)
    noise3 = noise.reshape(b, 1, n)
    out, mask3 = pl.pallas_call(
        functools.partial(_mask_kernel, n=n, num_masked=num_masked),
        grid=(b // bc,),
        in_specs=[
            pl.BlockSpec((bc, 1, n), lambda i: (i, 0, 0)),
            pl.BlockSpec((bc, n, p, d), lambda i: (i, 0, 0, 0)),
        ],
        out_specs=[
            pl.BlockSpec((bc, n, p, d), lambda i: (i, 0, 0, 0)),
            pl.BlockSpec((bc, 1, n), lambda i: (i, 0, 0)),
        ],
        out_shape=[
            jax.ShapeDtypeStruct((b, n, p, d), x.dtype),
            jax.ShapeDtypeStruct((b, 1, n), jnp.int32),
        ],
    )(noise3, x)
    return out, mask3.reshape(b, n).astype(bool)
